# baseline (speedup 1.0000x reference)
import functools

import jax
import jax.numpy as jnp
from jax import lax
from jax.experimental import pallas as pl
from jax.experimental.pallas import tpu as pltpu
from jax.experimental.pallas import tpu_sc as plsc

B = 4096
L = 200
EMBED = 32
POS = 200
LANES = 16

NUM_CORES = 2
NUM_SUBCORES = 16
NW = NUM_CORES * NUM_SUBCORES
BPW = B // NW
NPH = 4
PB = BPW // NPH
C0 = 104
C1 = L - C0
NSLOT = 4


def _sc_pool(x1t, x2t, x3t, x4t, e1, e2):
  mesh = plsc.VectorSubcoreMesh(core_axis_name="c", subcore_axis_name="s")

  @functools.partial(
      pl.kernel,
      mesh=mesh,
      compiler_params=pltpu.CompilerParams(
          use_tc_tiling_on_sc=False, needs_layout_passes=False),
      out_type=(
          jax.ShapeDtypeStruct((B, EMBED), jnp.float32),
          jax.ShapeDtypeStruct((B, EMBED), jnp.float32),
          jax.ShapeDtypeStruct((B,), jnp.float32),
      ),
      scratch_types=[
          pltpu.VMEM((L, PB), jnp.int32),
          pltpu.VMEM((L, PB), jnp.int32),
          pltpu.VMEM((L, PB), jnp.int32),
          pltpu.VMEM((L, PB), jnp.int32),
          pltpu.VMEM((L, PB), jnp.float32),
          pltpu.VMEM((L, PB), jnp.float32),
          pltpu.VMEM((PB * L,), jnp.int32),
          pltpu.VMEM((PB * L,), jnp.int32),
          pltpu.VMEM((NSLOT, L, EMBED), jnp.float32),
          pltpu.VMEM((NSLOT, L, EMBED), jnp.float32),
          pltpu.VMEM((POS + 1, 1), jnp.float32),
          pltpu.VMEM((PB, EMBED), jnp.float32),
          pltpu.VMEM((PB, EMBED), jnp.float32),
          pltpu.VMEM((PB,), jnp.float32),
          pltpu.SemaphoreType.DMA,
          pltpu.SemaphoreType.DMA,
          pltpu.SemaphoreType.DMA,
          pltpu.SemaphoreType.DMA,
          pltpu.SemaphoreType.DMA,
      ],
  )
  def pool(x1h, x2h, x3h, x4h, e1h, e2h, s1h, s2h, s3h,
           x1s, x2s, x3sa, x3sb, x4sa, x4sb, x1c, x2c, rows1, rows2,
           e2v, s1a, s2a, s3a, g0sem, g1sem, g2sem, g3sem, stsem):
    wid = lax.axis_index("s") * NUM_CORES + lax.axis_index("c")
    pltpu.sync_copy(e2h, e2v)
    lane = lax.iota(jnp.int32, LANES)
    zeros_i = jnp.zeros((LANES,), jnp.int32)
    zf = jnp.zeros((LANES,), jnp.float32)
    gsems = (g0sem, g1sem, g2sem, g3sem)
    x3bufs = (x3sa, x3sb)
    x4bufs = (x4sa, x4sb)

    def stage_copies(ph, x3buf, x4buf):
      base = wid * BPW + ph * PB
      return [
          pltpu.make_async_copy(x1h.at[:, pl.ds(base, PB)], x1s, stsem),
          pltpu.make_async_copy(x2h.at[:, pl.ds(base, PB)], x2s, stsem),
          pltpu.make_async_copy(x3h.at[:, pl.ds(base, PB)], x3buf, stsem),
          pltpu.make_async_copy(x4h.at[:, pl.ds(base, PB)], x4buf, stsem),
      ]

    for cp in stage_copies(0, x3sa, x4sa):
      cp.start()

    for ph in range(NPH):
      base = wid * BPW + ph * PB
      x3s = x3bufs[ph % 2]
      x4s = x4bufs[ph % 2]
      for cp in stage_copies(ph, x3s, x4s):
        cp.wait()

      col_starts = tuple(range(0, L - LANES, LANES)) + (L - LANES,)

      def tr_body(gi, carry):
        gvec = jnp.full((LANES,), gi, jnp.int32)
        for k in col_starts:
          kvec = k + lane
          v1 = plsc.load_gather(x1s, [kvec, gvec])
          v2 = plsc.load_gather(x2s, [kvec, gvec])
          off = pl.multiple_of(gi * L + k, 8)
          x1c[pl.ds(off, LANES)] = v1
          x2c[pl.ds(off, LANES)] = v2
        return carry

      lax.fori_loop(0, PB, tr_body, 0)

      if ph + 1 < NPH:
        for cp in stage_copies(ph + 1, x3bufs[(ph + 1) % 2],
                               x4bufs[(ph + 1) % 2]):
          cp.start()

      for g0 in range(0, PB, LANES):

        def t3_body(j, acc):
          pos = x3s[j, pl.ds(g0, LANES)]
          vals = plsc.load_gather(e2v, [pos, zeros_i])
          return acc + vals

        acc3 = lax.fori_loop(0, L, t3_body, zf, unroll=8)
        s3a[pl.ds(g0, LANES)] = acc3

      def chunk_copies(gi, slot):
        sem = gsems[slot]
        out = []
        for (idxc, rows) in ((x1c, rows1), (x2c, rows2)):
          off = pl.multiple_of(gi * L, 8)
          out.append(pltpu.make_async_copy(
              e1h.at[idxc.at[pl.ds(off, C0)]],
              rows.at[slot, pl.ds(0, C0)], sem))
          out.append(pltpu.make_async_copy(
              e1h.at[idxc.at[pl.ds(off + C0, C1)]],
              rows.at[slot, pl.ds(C0, C1)], sem))
        return out

      def fire(gi, slot):
        for cp in chunk_copies(gi, slot):
          cp.start()

      def drain(gi, slot):
        for cp in chunk_copies(gi, slot):
          cp.wait()

      def accumulate(gi, slot):
        def group(jb, jj_lo, accs):
          a10, a11, a20, a21 = accs
          w16 = plsc.load_gather(x4s, [jb + lane, jnp.full((LANES,), gi,
                                                           jnp.int32)])
          for jj in range(jj_lo, LANES):
            j = jb + jj
            w = jnp.take_along_axis(
                w16, jnp.full((LANES,), jj, jnp.int32), axis=0)
            a10 = a10 + rows1[slot, j, pl.ds(0, LANES)] * w
            a11 = a11 + rows1[slot, j, pl.ds(LANES, LANES)] * w
            a20 = a20 + rows2[slot, j, pl.ds(0, LANES)]
            a21 = a21 + rows2[slot, j, pl.ds(LANES, LANES)]
          return (a10, a11, a20, a21)

        def group_body(k, accs):
          return group(pl.multiple_of(k * LANES, LANES), 0, accs)

        accs = lax.fori_loop(0, L // LANES, group_body, (zf, zf, zf, zf))
        a10, a11, a20, a21 = group(L - LANES, LANES - (L % LANES), accs)
        s1a[gi, pl.ds(0, LANES)] = a10
        s1a[gi, pl.ds(LANES, LANES)] = a11
        s2a[gi, pl.ds(0, LANES)] = a20
        s2a[gi, pl.ds(LANES, LANES)] = a21

      fire(0, 0)
      fire(1, 1)
      fire(2, 2)

      def quad_body(i, carry):
        for sl in range(NSLOT):
          gi = i * NSLOT + sl
          drain(gi, sl)

          @pl.when(gi + 3 < PB)
          def _():
            fire(gi + 3, (sl + 3) % NSLOT)

          accumulate(gi, sl)
        return carry

      lax.fori_loop(0, PB // NSLOT, quad_body, 0)

      pltpu.sync_copy(s1a, s1h.at[pl.ds(base, PB)])
      pltpu.sync_copy(s2a, s2h.at[pl.ds(base, PB)])
      pltpu.sync_copy(s3a, s3h.at[pl.ds(base, PB)])

  return pool(x1t, x2t, x3t, x4t, e1, e2)


def _tc_tail(s1, s2, s3, t1b, w1, bb1, t2b, w2, bb2):
  def body(s1r, s2r, s3r, t1br, w1r, b1r, t2br, w2r, b2r, outr):
    h1 = jnp.tanh(s1r[...] + t1br[...][None, :])
    h1 = jnp.tanh(
        lax.dot_general(h1, w1r[...], (((1,), (1,)), ((), ())),
                        preferred_element_type=jnp.float32) + b1r[...][None, :])
    h2 = jnp.tanh(s2r[...] + t2br[...][None, :])
    h2 = jnp.tanh(
        lax.dot_general(h2, w2r[...], (((1,), (1,)), ((), ())),
                        preferred_element_type=jnp.float32) + b2r[...][None, :])
    x12 = jax.nn.sigmoid(jnp.sum(h1 * h2, axis=1))
    outr[...] = x12 * jax.nn.sigmoid(s3r[...])

  return pl.pallas_call(
      body,
      out_shape=jax.ShapeDtypeStruct((B,), jnp.float32),
  )(s1, s2, s3, t1b, w1, bb1, t2b, w2, bb2)


def kernel(x1, x2, x3, x4, E1, t1_bias1, W1, b1, t2_bias1, W2, b2, E2):
  s1, s2, s3 = _sc_pool(x1.astype(jnp.int32).T, x2.astype(jnp.int32).T,
                        x3.astype(jnp.int32).T, x4.T, E1, E2)
  return _tc_tail(s1, s2, s3, t1_bias1, W1, b1, t2_bias1, W2, b2)
